# Initial kernel scaffold; baseline (speedup 1.0000x reference)
import jax
import jax.numpy as jnp
from jax.experimental import pallas as pl


def kernel(x, embeds, gene_proj_w, gene_proj_b, ge_w, ge_b, out_w, out_b):
    raise NotImplementedError("write your pallas kernel here")


# speedup vs baseline: 3.4111x; 3.4111x over previous
import jax
import jax.numpy as jnp
from jax.experimental import pallas as pl
from jax.experimental.pallas import tpu as pltpu

NUM_NODES = 1024
EMBED_DIM = 128
MS_SIGMA = 1.0
MS_DAMPING = 0.5
MS_MAX_ITER = 10
MS_ALPHA = 10.0
BATCH = 32


def _dot(a, b):
    return jnp.dot(a, b, preferred_element_type=jnp.float32)


def _ted_body(x_ref, emb_ref, gpw_ref, gpb_ref, gew_ref, geb_ref,
              ow_ref, ob_ref, out_ref):
    P = emb_ref[...]
    p_sq = jnp.sum(P * P, axis=1)
    inv2s2 = 1.0 / (2.0 * MS_SIGMA * MS_SIGMA)

    Y = P
    for _ in range(MS_MAX_ITER):
        y_sq = jnp.sum(Y * Y, axis=1)
        d2 = y_sq[:, None] + p_sq[None, :] - 2.0 * _dot(Y, P.T)
        d2 = jnp.maximum(d2, 0.0)
        w = jnp.exp(-d2 * inv2s2)
        Y_new = _dot(w, P) / (jnp.sum(w, axis=1, keepdims=True) + 1e-8)
        Y = MS_DAMPING * Y + (1.0 - MS_DAMPING) * Y_new

    y_sq = jnp.sum(Y * Y, axis=1)
    d2 = p_sq[:, None] + y_sq[None, :] - 2.0 * _dot(P, Y.T)
    d2 = jnp.maximum(d2, 0.0)
    logits = -MS_ALPHA * d2
    logits = logits - jnp.max(logits, axis=1, keepdims=True)
    eL = jnp.exp(logits)
    H = eL / jnp.sum(eL, axis=1, keepdims=True)

    dv = jnp.sum(H, axis=1)
    de = jnp.sum(H, axis=0)
    Hn = H * jax.lax.rsqrt(dv + 1e-8)[:, None]
    inv_de = 1.0 / (de + 1e-8)

    W1 = gew_ref[0:EMBED_DIM, :]
    W2 = gew_ref[EMBED_DIM:2 * EMBED_DIM, :]
    u = _dot(gpw_ref[...], W1)
    F0 = _dot(P, W2) + _dot(gpb_ref[...], W1) + geb_ref[...]

    M0 = _dot(Hn.T, F0) * inv_de[:, None]
    C = _dot(Hn, M0)

    s = _dot(x_ref[...], Hn)
    t = _dot(s * inv_de[None, :], Hn.T)

    z = jnp.maximum(t[:, :, None] * u[0][None, None, :] + C[None, :, :], 0.0)
    r = jnp.sum(z * ow_ref[...][0][None, None, :], axis=2)
    energy = jnp.sum(r, axis=1) * (1.0 / NUM_NODES) + ob_ref[0, 0]
    out_ref[...] = energy[:, None]


def kernel(x, embeds, gene_proj_w, gene_proj_b, ge_w, ge_b, out_w, out_b):
    out = pl.pallas_call(
        _ted_body,
        out_shape=jax.ShapeDtypeStruct((BATCH, 1), jnp.float32),
    )(
        x,
        embeds,
        gene_proj_w,
        gene_proj_b.reshape(1, EMBED_DIM),
        ge_w,
        ge_b.reshape(1, EMBED_DIM),
        out_w.reshape(1, EMBED_DIM),
        out_b.reshape(1, 1),
    )
    return out[:, 0]


# speedup vs baseline: 3.6255x; 1.0629x over previous
import jax
import jax.numpy as jnp
from jax.experimental import pallas as pl
from jax.experimental.pallas import tpu as pltpu

NUM_NODES = 1024
EMBED_DIM = 128
MS_SIGMA = 1.0
MS_DAMPING = 0.5
MS_MAX_ITER = 10
MS_ALPHA = 10.0
BATCH = 32


def _dot(a, b):
    return jnp.dot(a, b, preferred_element_type=jnp.float32)


def _bf(a):
    return a.astype(jnp.bfloat16)


def _ted_body(x_ref, emb_ref, gpw_ref, gpb_ref, gew_ref, geb_ref,
              ow_ref, ob_ref, out_ref):
    P = emb_ref[...]
    p_sq = jnp.sum(P * P, axis=1)
    inv2s2 = 1.0 / (2.0 * MS_SIGMA * MS_SIGMA)

    Pb = _bf(P)
    Pbt = Pb.T
    hp = inv2s2 * p_sq
    Y = P
    for _ in range(MS_MAX_ITER):
        hy = inv2s2 * jnp.sum(Y * Y, axis=1)
        G = _dot(_bf(2.0 * inv2s2 * Y), Pbt)
        w = jnp.exp(jnp.minimum(G - hy[:, None] - hp[None, :], 0.0))
        Y_new = _dot(_bf(w), Pb) / (jnp.sum(w, axis=1, keepdims=True) + 1e-8)
        Y = MS_DAMPING * Y + (1.0 - MS_DAMPING) * Y_new

    ay = MS_ALPHA * jnp.sum(Y * Y, axis=1)
    G = _dot(Pb, _bf(2.0 * MS_ALPHA * Y).T)
    logits = jnp.minimum(G - MS_ALPHA * p_sq[:, None] - ay[None, :], 0.0)
    logits = logits - jnp.max(logits, axis=1, keepdims=True)
    eL = jnp.exp(logits)
    H = eL / jnp.sum(eL, axis=1, keepdims=True)

    dv = jnp.sum(H, axis=1)
    de = jnp.sum(H, axis=0)
    Hn = H * jax.lax.rsqrt(dv + 1e-8)[:, None]
    inv_de = 1.0 / (de + 1e-8)

    W1 = gew_ref[0:EMBED_DIM, :]
    W2 = gew_ref[EMBED_DIM:2 * EMBED_DIM, :]
    u = _dot(gpw_ref[...], W1)
    F0 = _dot(P, W2) + _dot(gpb_ref[...], W1) + geb_ref[...]

    M0 = _dot(Hn.T, F0) * inv_de[:, None]
    C = _dot(Hn, M0)

    s = _dot(x_ref[...], Hn)
    t = _dot(s * inv_de[None, :], Hn.T)

    z = jnp.maximum(t[:, :, None] * u[0][None, None, :] + C[None, :, :], 0.0)
    r = jnp.sum(z * ow_ref[...][0][None, None, :], axis=2)
    energy = jnp.sum(r, axis=1) * (1.0 / NUM_NODES) + ob_ref[0, 0]
    out_ref[...] = energy[:, None]


def kernel(x, embeds, gene_proj_w, gene_proj_b, ge_w, ge_b, out_w, out_b):
    out = pl.pallas_call(
        _ted_body,
        out_shape=jax.ShapeDtypeStruct((BATCH, 1), jnp.float32),
    )(
        x,
        embeds,
        gene_proj_w,
        gene_proj_b.reshape(1, EMBED_DIM),
        ge_w,
        ge_b.reshape(1, EMBED_DIM),
        out_w.reshape(1, EMBED_DIM),
        out_b.reshape(1, 1),
    )
    return out[:, 0]
